# baseline (speedup 1.0000x reference)
import functools

import jax
import jax.numpy as jnp
from jax import lax
from jax.experimental import pallas as pl
from jax.experimental.pallas import tpu as pltpu
from jax.experimental.pallas import tpu_sc as plsc

NC = 2
NS = 16
NW = NC * NS
L = 16

K = 128
CH = 2048

_SC_PARAMS = pltpu.CompilerParams(
    needs_layout_passes=False, use_tc_tiling_on_sc=False)

_MESH = dict(core_axis_name="c", subcore_axis_name="s")


def _pad_to(n, m):
    return ((n + m - 1) // m) * m


def _make_deg_kernel(NP, CPT):
    NPT = NP // NS

    @functools.partial(
        pl.kernel,
        out_type=jax.ShapeDtypeStruct((NC, NP), jnp.float32),
        mesh=plsc.VectorSubcoreMesh(**_MESH),
        scratch_types=[
            pltpu.VMEM((CPT, K), jnp.int32),
            pltpu.VMEM((CPT, K), jnp.float32),
            pltpu.VMEM((K,), jnp.float32),
            pltpu.VMEM_SHARED((NP,), jnp.float32),
            pltpu.SemaphoreType.DMA,
        ],
        compiler_params=_SC_PARAMS,
    )
    def deg_kernel(dst_hbm, ew_hbm, out_hbm, dst_v, ew_v, buf_v, deg_sh, sem):
        c = lax.axis_index("c")
        s = lax.axis_index("s")
        wid = c * NS + s

        pltpu.sync_copy(dst_hbm.at[pl.ds(wid * CPT, CPT)], dst_v)
        pltpu.sync_copy(ew_hbm.at[pl.ds(wid * CPT, CPT)], ew_v)

        @pl.loop(0, K // L)
        def _zero(q):
            buf_v[pl.ds(q * L, L)] = jnp.zeros((L,), jnp.float32)
        for j in range(NPT // K):
            pltpu.sync_copy(buf_v, deg_sh.at[pl.ds(s * NPT + j * K, K)])
        plsc.subcore_barrier()

        @pl.loop(0, CPT)
        def _fire(j):
            pltpu.async_copy(ew_v.at[j], deg_sh.at[dst_v.at[j]], sem,
                             add=True)

        @pl.loop(0, CPT)
        def _drain(j):
            pltpu.make_async_copy(ew_v.at[j], deg_sh.at[dst_v.at[j]],
                                  sem).wait()

        plsc.subcore_barrier()
        for j in range(NPT // K):
            off = s * NPT + j * K
            pltpu.sync_copy(deg_sh.at[pl.ds(off, K)], buf_v)
            pltpu.sync_copy(buf_v, out_hbm.at[c, pl.ds(off, K)])

    return deg_kernel


def _make_agg_kernel(NP, EPC, F):
    FPT = F // NS
    NCH = EPC // CH

    @functools.partial(
        pl.kernel,
        out_type=jax.ShapeDtypeStruct((NC, F, NP), jnp.float32),
        mesh=plsc.VectorSubcoreMesh(**_MESH),
        scratch_types=[
            pltpu.VMEM((CH,), jnp.int32),
            pltpu.VMEM((CH,), jnp.int32),
            pltpu.VMEM((CH,), jnp.float32),
            pltpu.VMEM((CH,), jnp.int32),
            pltpu.VMEM((CH,), jnp.int32),
            pltpu.VMEM((CH,), jnp.float32),
            pltpu.VMEM((FPT, NP), jnp.float32),
            pltpu.VMEM((FPT, NP), jnp.float32),
            pltpu.SemaphoreType.DMA,
            pltpu.SemaphoreType.DMA,
        ],
        compiler_params=_SC_PARAMS,
    )
    def agg_kernel(src_hbm, dst_hbm, ew_hbm, ht_hbm, out_hbm,
                   src_a, dst_a, ew_a, src_b, dst_b, ew_b,
                   h_t, acc, sem_a, sem_b):
        c = lax.axis_index("c")
        s = lax.axis_index("s")
        base = c * EPC
        bufs = ((src_a, dst_a, ew_a, sem_a), (src_b, dst_b, ew_b, sem_b))

        pltpu.sync_copy(ht_hbm.at[pl.ds(s * FPT, FPT)], h_t)

        for f in range(FPT):
            @pl.loop(0, NP // L)
            def _zero(q):
                acc[f, pl.ds(q * L, L)] = jnp.zeros((L,), jnp.float32)

        def start(ch, b):
            sv, dv, ev, sem = bufs[b]
            off = base + ch * CH
            pltpu.async_copy(src_hbm.at[pl.ds(off, CH)], sv, sem)
            pltpu.async_copy(dst_hbm.at[pl.ds(off, CH)], dv, sem)
            pltpu.async_copy(ew_hbm.at[pl.ds(off, CH)], ev, sem)

        def wait(ch, b):
            sv, dv, ev, sem = bufs[b]
            off = base + ch * CH
            pltpu.make_async_copy(src_hbm.at[pl.ds(off, CH)], sv, sem).wait()
            pltpu.make_async_copy(dst_hbm.at[pl.ds(off, CH)], dv, sem).wait()
            pltpu.make_async_copy(ew_hbm.at[pl.ds(off, CH)], ev, sem).wait()

        def proc(b):
            sv, dv, ev, _ = bufs[b]

            @pl.loop(0, CH // (2 * L))
            def _grp(g):
                for u in range(2):
                    off = (g * 2 + u) * L
                    src16 = sv[pl.ds(off, L)]
                    dst16 = dv[pl.ds(off, L)]
                    ew16 = ev[pl.ds(off, L)]
                    for f in range(FPT):
                        fi = jnp.full((L,), f, jnp.int32)
                        v = plsc.load_gather(h_t, [fi, src16])
                        plsc.addupdate_scatter(acc, [fi, dst16], v * ew16)

        start(0, 0)
        start(1, 1)

        @pl.loop(0, NCH // 2)
        def _pair(p):
            for b in range(2):
                ch = 2 * p + b
                wait(ch, b)
                proc(b)

                @pl.when(ch + 2 < NCH)
                def _():
                    start(ch + 2, b)

        pltpu.sync_copy(acc, out_hbm.at[c, pl.ds(s * FPT, FPT)])

    return agg_kernel


def _tc1_body(deg_ref, x_ref, w_ref, dis_ref, h_ref):
    deg = deg_ref[0:1, :] + deg_ref[1:2, :]
    safe = jnp.where(deg > 0, deg, 1.0)
    dis = jnp.where(deg > 0, lax.rsqrt(safe), 0.0)
    dis_ref[...] = dis
    ht = lax.dot_general(
        w_ref[...], x_ref[...], (((0,), (1,)), ((), ())),
        preferred_element_type=jnp.float32,
        precision=lax.Precision.HIGHEST)
    h_ref[...] = ht * dis


def _tc2_body(p_ref, dis_ref, b_ref, w_ref, h_ref):
    dis = dis_ref[...]
    z = (p_ref[0] + p_ref[1]) * dis + b_ref[...]
    z = jnp.maximum(z, 0.0)
    ht = lax.dot_general(
        w_ref[...], z, (((0,), (0,)), ((), ())),
        preferred_element_type=jnp.float32,
        precision=lax.Precision.HIGHEST)
    h_ref[...] = ht * dis


def _tc3_body(p_ref, dis_ref, b_ref, logits_ref, soft_ref):
    lt = (p_ref[0] + p_ref[1]) * dis_ref[...] + b_ref[...]
    logits_ref[...] = lt.T
    m = jnp.max(lt, axis=0, keepdims=True)
    e = jnp.exp(lt - m)
    soft_ref[...] = (e / jnp.sum(e, axis=0, keepdims=True)).T


def kernel(x, edge_index, edge_weight, W1, b1, W2, b2):
    N, D = x.shape
    H = W1.shape[1]
    C = W2.shape[1]
    E = edge_index.shape[1]

    NP = _pad_to(N, NS * L * NC)
    E_pad = _pad_to(E, NC * 2 * CH)
    E_pad = _pad_to(E_pad, NW * K)
    EPC = E_pad // NC

    src = jnp.pad(edge_index[0].astype(jnp.int32), (0, E_pad - E))
    dst = jnp.pad(edge_index[1].astype(jnp.int32), (0, E_pad - E))
    ew = jnp.pad(edge_weight, (0, E_pad - E))
    dst2 = dst.reshape(E_pad // K, K)
    ew2 = ew.reshape(E_pad // K, K)
    x_pad = jnp.pad(x, ((0, NP - N), (0, 0)))

    CPT = E_pad // NW // K
    deg2 = _make_deg_kernel(NP, CPT)(dst2, ew2)

    dis, h1t = pl.pallas_call(
        _tc1_body,
        out_shape=(jax.ShapeDtypeStruct((1, NP), jnp.float32),
                   jax.ShapeDtypeStruct((H, NP), jnp.float32)),
    )(deg2, x_pad, W1)

    agg1 = _make_agg_kernel(NP, EPC, H)(src, dst, ew, h1t)

    h2t = pl.pallas_call(
        _tc2_body,
        out_shape=jax.ShapeDtypeStruct((C, NP), jnp.float32),
    )(agg1, dis, b1.reshape(H, 1), W2)

    agg2 = _make_agg_kernel(NP, EPC, C)(src, dst, ew, h2t)

    logits, soft = pl.pallas_call(
        _tc3_body,
        out_shape=(jax.ShapeDtypeStruct((NP, C), jnp.float32),
                   jax.ShapeDtypeStruct((NP, C), jnp.float32)),
    )(agg2, dis, b2.reshape(C, 1))

    return logits[:N], soft[:N]


# speedup vs baseline: 1.7952x; 1.7952x over previous
import functools

import jax
import jax.numpy as jnp
from jax import lax
from jax.experimental import pallas as pl
from jax.experimental.pallas import tpu as pltpu
from jax.experimental.pallas import tpu_sc as plsc

NC = 2
NS = 16
NW = NC * NS
L = 16

K = 128

_GDN = lax.GatherDimensionNumbers(
    offset_dims=(), collapsed_slice_dims=(0,), start_index_map=(0,))

_SC_PARAMS = pltpu.CompilerParams(
    needs_layout_passes=False, use_tc_tiling_on_sc=False)

_MESH = dict(core_axis_name="c", subcore_axis_name="s")


def _pad_to(n, m):
    return ((n + m - 1) // m) * m


def _lane_bcast(vec, r):
    idx = jnp.full((L, 1), r, dtype=jnp.int32)
    return lax.gather(vec, idx, _GDN, slice_sizes=(1,),
                      mode=lax.GatherScatterMode.PROMISE_IN_BOUNDS)


def _make_deg_kernel(NP, CPT):
    NPT = NP // NS

    @functools.partial(
        pl.kernel,
        out_type=jax.ShapeDtypeStruct((NC, NP), jnp.float32),
        mesh=plsc.VectorSubcoreMesh(**_MESH),
        scratch_types=[
            pltpu.VMEM((CPT, K), jnp.int32),
            pltpu.VMEM((CPT, K), jnp.float32),
            pltpu.VMEM((K,), jnp.float32),
            pltpu.VMEM_SHARED((NP,), jnp.float32),
            pltpu.SemaphoreType.DMA,
        ],
        compiler_params=_SC_PARAMS,
    )
    def deg_kernel(dst_hbm, ew_hbm, out_hbm, dst_v, ew_v, buf_v, deg_sh, sem):
        c = lax.axis_index("c")
        s = lax.axis_index("s")
        wid = c * NS + s

        pltpu.sync_copy(dst_hbm.at[pl.ds(wid * CPT, CPT)], dst_v)
        pltpu.sync_copy(ew_hbm.at[pl.ds(wid * CPT, CPT)], ew_v)

        @pl.loop(0, K // L)
        def _zero(q):
            buf_v[pl.ds(q * L, L)] = jnp.zeros((L,), jnp.float32)
        for j in range(NPT // K):
            pltpu.sync_copy(buf_v, deg_sh.at[pl.ds(s * NPT + j * K, K)])
        plsc.subcore_barrier()

        @pl.loop(0, CPT)
        def _fire(j):
            pltpu.async_copy(ew_v.at[j], deg_sh.at[dst_v.at[j]], sem,
                             add=True)

        @pl.loop(0, CPT)
        def _drain(j):
            pltpu.make_async_copy(ew_v.at[j], deg_sh.at[dst_v.at[j]],
                                  sem).wait()

        plsc.subcore_barrier()
        for j in range(NPT // K):
            off = s * NPT + j * K
            pltpu.sync_copy(deg_sh.at[pl.ds(off, K)], buf_v)
            pltpu.sync_copy(buf_v, out_hbm.at[c, pl.ds(off, K)])

    return deg_kernel


def _make_agg_kernel(NP, CPT, G, F):
    SUP = G * K
    RPT = NP // NS
    NSUP = CPT // G
    NB = 3
    NP3 = -(-NSUP // NB)

    @functools.partial(
        pl.kernel,
        out_type=jax.ShapeDtypeStruct((NC, NP, F), jnp.float32),
        mesh=plsc.VectorSubcoreMesh(**_MESH),
        scratch_types=[
            pltpu.VMEM((CPT, K), jnp.int32),
            pltpu.VMEM((CPT, K), jnp.int32),
            pltpu.VMEM((CPT, K), jnp.float32),
            pltpu.VMEM((SUP, F), jnp.bfloat16),
            pltpu.VMEM((SUP, F), jnp.bfloat16),
            pltpu.VMEM((SUP, F), jnp.bfloat16),
            pltpu.VMEM((SUP, F), jnp.float32),
            pltpu.VMEM((SUP, F), jnp.float32),
            pltpu.VMEM((SUP, F), jnp.float32),
            pltpu.VMEM_SHARED((NP, F), jnp.float32),
            pltpu.SemaphoreType.DMA,
            pltpu.SemaphoreType.DMA,
            pltpu.SemaphoreType.DMA,
            pltpu.SemaphoreType.DMA,
            pltpu.SemaphoreType.DMA,
            pltpu.SemaphoreType.DMA,
        ],
        compiler_params=_SC_PARAMS,
    )
    def agg_kernel(src_hbm, dst_hbm, ew_hbm, h_hbm, out_hbm,
                   src_v, dst_v, ew_v, rows0, rows1, rows2,
                   scat0, scat1, scat2,
                   agg_sh, gsem0, gsem1, gsem2, ssem0, ssem1, ssem2):
        c = lax.axis_index("c")
        s = lax.axis_index("s")
        wid = c * NS + s
        bufs = ((rows0, scat0, gsem0, ssem0), (rows1, scat1, gsem1, ssem1),
                (rows2, scat2, gsem2, ssem2))

        pltpu.sync_copy(src_hbm.at[pl.ds(wid * CPT, CPT)], src_v)
        pltpu.sync_copy(dst_hbm.at[pl.ds(wid * CPT, CPT)], dst_v)
        pltpu.sync_copy(ew_hbm.at[pl.ds(wid * CPT, CPT)], ew_v)

        @pl.loop(0, K)
        def _zero(r):
            for f in range(F // L):
                scat0[r, pl.ds(f * L, L)] = jnp.zeros((L,), jnp.float32)
        for m in range(RPT // K):
            pltpu.sync_copy(scat0.at[pl.ds(0, K)],
                            agg_sh.at[pl.ds(s * RPT + m * K, K)])
        plsc.subcore_barrier()

        def gather(t, b):
            rowsb, _, gsem, _ = bufs[b]
            for j in range(G):
                pltpu.async_copy(h_hbm.at[src_v.at[t * G + j]],
                                 rowsb.at[pl.ds(j * K, K)], gsem)

        def proc(t, b):
            rowsb, scatb, gsem, ssem = bufs[b]
            for j in range(G):
                pltpu.make_async_copy(h_hbm.at[src_v.at[t * G + j]],
                                      rowsb.at[pl.ds(j * K, K)], gsem).wait()
            for j in range(G):
                @pl.loop(0, K // L)
                def _scale(q):
                    w16 = ew_v[t * G + j, pl.ds(q * L, L)]
                    for r in range(L):
                        bc = _lane_bcast(w16, r)
                        row = j * K + q * L + r
                        for f in range(F // 32):
                            v32 = rowsb[row, pl.ds(f * 32, 32)]
                            v = plsc.bitcast(v32, jnp.int32)
                            lo = plsc.bitcast(v << 16, jnp.float32)
                            hi = plsc.bitcast(
                                v & jnp.int32(-65536), jnp.float32)
                            scatb[row, pl.ds(2 * f * L, L)] = lo * bc
                            scatb[row, pl.ds((2 * f + 1) * L, L)] = hi * bc

                pltpu.async_copy(scatb.at[pl.ds(j * K, K)],
                                 agg_sh.at[dst_v.at[t * G + j]], ssem,
                                 add=True)

        def drain_scat(t, b):
            _, scatb, _, ssem = bufs[b]
            for j in range(G):
                pltpu.make_async_copy(scatb.at[pl.ds(j * K, K)],
                                      agg_sh.at[dst_v.at[t * G + j]],
                                      ssem).wait()

        gather(0, 0)

        @pl.loop(0, NP3)
        def _pipe(p):
            for i in range(NB):
                t = NB * p + i
                nxt = (i + 1) % NB

                @pl.when(jnp.logical_and(t + 1 < NSUP, t >= NB - 1))
                def _():
                    drain_scat(t + 1 - NB, nxt)

                @pl.when(t + 1 < NSUP)
                def _():
                    gather(t + 1, nxt)

                @pl.when(t < NSUP)
                def _():
                    proc(t, i)

        for i in range(min(NB, NSUP)):
            t = NSUP - 1 - i
            drain_scat(t, t % NB)
        plsc.subcore_barrier()
        for m in range(RPT // K):
            off = s * RPT + m * K
            pltpu.sync_copy(agg_sh.at[pl.ds(off, K)], scat0.at[pl.ds(0, K)])
            pltpu.sync_copy(scat0.at[pl.ds(0, K)], out_hbm.at[c, pl.ds(off, K)])

    return agg_kernel


def _feat_perm(F):
    return jnp.array([32 * c + 16 * j + k
                      for c in range(F // 32)
                      for k in range(16)
                      for j in range(2)], dtype=jnp.int32)


def _tc1_body(deg_ref, x_ref, w_ref, dis_ref, h_ref):
    deg = deg_ref[:, 0:1] + deg_ref[:, 1:2]
    safe = jnp.where(deg > 0, deg, 1.0)
    dis = jnp.where(deg > 0, lax.rsqrt(safe), 0.0)
    dis_ref[...] = dis
    h = jnp.dot(x_ref[...], w_ref[...], preferred_element_type=jnp.float32,
                precision=lax.Precision.HIGHEST)
    h_ref[...] = (h * dis).astype(jnp.bfloat16)


def _tc2_body(p_ref, dis_ref, b_ref, w_ref, h_ref):
    dis = dis_ref[...]
    z = (p_ref[0] + p_ref[1]) * dis + b_ref[...]
    z = jnp.maximum(z, 0.0)
    h = jnp.dot(z, w_ref[...], preferred_element_type=jnp.float32,
                precision=lax.Precision.HIGHEST)
    h_ref[...] = (h * dis).astype(jnp.bfloat16)


def _tc3_body(p_ref, dis_ref, b_ref, logits_ref, soft_ref):
    logits = (p_ref[0] + p_ref[1]) * dis_ref[...] + b_ref[...]
    logits_ref[...] = logits
    m = jnp.max(logits, axis=1, keepdims=True)
    e = jnp.exp(logits - m)
    soft_ref[...] = e / jnp.sum(e, axis=1, keepdims=True)


def kernel(x, edge_index, edge_weight, W1, b1, W2, b2):
    N, D = x.shape
    H = W1.shape[1]
    C = W2.shape[1]
    E = edge_index.shape[1]

    NP = _pad_to(N, NS * L * NC)
    SUPER = 1024
    EPT = _pad_to(-(-E // NW), 2 * SUPER)
    E_pad = EPT * NW

    src = jnp.pad(edge_index[0].astype(jnp.int32), (0, E_pad - E))
    dst = jnp.pad(edge_index[1].astype(jnp.int32), (0, E_pad - E))
    ew = jnp.pad(edge_weight, (0, E_pad - E))
    src2 = src.reshape(E_pad // K, K)
    dst2 = dst.reshape(E_pad // K, K)
    ew2 = ew.reshape(E_pad // K, K)
    x_pad = jnp.pad(x, ((0, NP - N), (0, 0)))

    CPT = EPT // K
    deg2 = _make_deg_kernel(NP, CPT)(dst2, ew2)
    deg2t = deg2.T

    dis, h1p = pl.pallas_call(
        _tc1_body,
        out_shape=(jax.ShapeDtypeStruct((NP, 1), jnp.float32),
                   jax.ShapeDtypeStruct((NP, H), jnp.bfloat16)),
    )(deg2t, x_pad, W1[:, _feat_perm(H)])

    agg1 = _make_agg_kernel(NP, CPT, 1, H)(src2, dst2, ew2, h1p)

    h2p = pl.pallas_call(
        _tc2_body,
        out_shape=jax.ShapeDtypeStruct((NP, C), jnp.bfloat16),
    )(agg1, dis, b1.reshape(1, H), W2[:, _feat_perm(C)])

    agg2 = _make_agg_kernel(NP, CPT, 4, C)(src2, dst2, ew2, h2p)

    logits, soft = pl.pallas_call(
        _tc3_body,
        out_shape=(jax.ShapeDtypeStruct((NP, C), jnp.float32),
                   jax.ShapeDtypeStruct((NP, C), jnp.float32)),
    )(agg2, dis, b2.reshape(1, C))

    return logits[:N], soft[:N]
